# baseline (speedup 1.0000x reference)
import functools

import jax
import jax.numpy as jnp
from jax import lax
from jax.experimental import pallas as pl
from jax.experimental.pallas import tpu as pltpu
from jax.experimental.pallas import tpu_sc as plsc

_NBUF = 4


def _sc_gather_kernel(n_total, d_model, b_per_w, chunk, n_chunks):
    mesh = plsc.VectorSubcoreMesh(core_axis_name="c", subcore_axis_name="s")

    @functools.partial(
        pl.kernel,
        mesh=mesh,
        out_type=jax.ShapeDtypeStruct((n_total, d_model), jnp.float32),
        scratch_types=[
            pltpu.VMEM((n_chunks, chunk), jnp.int32),
            pltpu.VMEM((_NBUF, chunk, d_model), jnp.float32),
            pltpu.SemaphoreType.DMA((_NBUF,)),
            pltpu.SemaphoreType.DMA((_NBUF,)),
        ],
    )
    def k(table_hbm, idx_hbm, out_hbm, idx_v, rows_v, gsem, ssem):
        nc = plsc.get_sparse_core_info().num_cores
        wid = lax.axis_index("s") * nc + lax.axis_index("c")
        base = wid * b_per_w
        pltpu.sync_copy(idx_hbm.at[wid], idx_v)

        def gather(c, b):
            pltpu.make_async_copy(
                table_hbm.at[idx_v.at[c]], rows_v.at[b], gsem.at[b]
            ).start()

        def scatter(c, b):
            pltpu.make_async_copy(
                table_hbm.at[idx_v.at[c]], rows_v.at[b], gsem.at[b]
            ).wait()
            pltpu.make_async_copy(
                rows_v.at[b],
                out_hbm.at[pl.ds(base + c * chunk, chunk)],
                ssem.at[b],
            ).start()

        def wait_scatter(c, b):
            pltpu.make_async_copy(
                rows_v.at[b],
                out_hbm.at[pl.ds(base + c * chunk, chunk)],
                ssem.at[b],
            ).wait()

        for c in range(_NBUF):
            gather(c, c)
        scatter(0, 0)

        n_steady = n_chunks - _NBUF
        assert n_steady % _NBUF == 0

        def body(j):
            c0 = 1 + j * _NBUF
            for u in range(_NBUF):
                c = c0 + u
                scatter(c, (1 + u) % _NBUF)
                wait_scatter(c - 1, u % _NBUF)
                gather(c + 3, u % _NBUF)

        pl.loop(0, n_steady // _NBUF)(body)

        for c in range(n_chunks - 3, n_chunks):
            scatter(c, c % _NBUF)
            wait_scatter(c - 1, (c - 1) % _NBUF)
        wait_scatter(n_chunks - 1, (n_chunks - 1) % _NBUF)

    return k


def _tc_trig_kernel(n_rows, d_model, blk, n_hi, n_lo):
    n_blocks = n_rows // blk

    def body(idx_ref, phi_ref, qlo_ref, pthi_ref, qtlo_ref, out_ref):
        idx = idx_ref[0, 0, :]
        hi = idx >> 6
        lo = idx & 63
        ih = lax.broadcasted_iota(jnp.int32, (blk, n_hi), 1)
        il = lax.broadcasted_iota(jnp.int32, (blk, n_lo), 1)
        oh_hi = (hi[:, None] == ih).astype(jnp.bfloat16)
        oh_lo = (lo[:, None] == il).astype(jnp.bfloat16)
        a = jnp.dot(oh_hi, phi_ref[...], preferred_element_type=jnp.float32)
        b = jnp.dot(oh_lo, qlo_ref[...], preferred_element_type=jnp.float32)
        c = jnp.dot(oh_hi, pthi_ref[...], preferred_element_type=jnp.float32)
        d = jnp.dot(oh_lo, qtlo_ref[...], preferred_element_type=jnp.float32)
        out_ref[...] = a * b + c * d

    grid_spec = pl.GridSpec(
        grid=(n_blocks,),
        in_specs=[
            pl.BlockSpec((1, 1, blk), lambda i: (i, 0, 0)),
            pl.BlockSpec((n_hi, d_model), lambda i: (0, 0)),
            pl.BlockSpec((n_lo, d_model), lambda i: (0, 0)),
            pl.BlockSpec((n_hi, d_model), lambda i: (0, 0)),
            pl.BlockSpec((n_lo, d_model), lambda i: (0, 0)),
        ],
        out_specs=pl.BlockSpec((blk, d_model), lambda i: (i, 0)),
    )
    return pl.pallas_call(
        body,
        grid_spec=grid_spec,
        out_shape=jax.ShapeDtypeStruct((n_rows, d_model), jnp.float32),
    )


def _trig_tables(pe):
    v, d = pe.shape
    hi_rows = pe[::64]
    lo_rows = pe[:64]
    sin_hi = hi_rows[:, 0::2]
    cos_hi = hi_rows[:, 1::2]
    sin_lo = lo_rows[:, 0::2]
    cos_lo = lo_rows[:, 1::2]

    def interleave(a, b):
        return jnp.stack([a, b], axis=-1).reshape(a.shape[0], d)

    p_hi = hi_rows
    q_lo = interleave(cos_lo, cos_lo)
    pt_hi = interleave(cos_hi, -sin_hi)
    qt_lo = interleave(sin_lo, sin_lo)
    bf = jnp.bfloat16
    return p_hi.astype(bf), q_lo.astype(bf), pt_hi.astype(bf), qt_lo.astype(bf)


def kernel(x, pe):
    b, s = x.shape
    v, d = pe.shape
    info = plsc.get_sparse_core_info()
    nw = info.num_cores * info.num_subcores

    b_sc = 2
    n_sc = b_sc * s
    n_tc = (b - b_sc) * s

    xi = x.astype(jnp.int32)

    b_per_w = n_sc // nw
    chunk = 16
    n_chunks = b_per_w // chunk
    idx3 = xi[:b_sc].reshape(nw, n_chunks, chunk)
    sc = _sc_gather_kernel(n_sc, d, b_per_w, chunk, n_chunks)
    out_sc = sc(pe, idx3)

    blk = 256
    idx_tc = xi[b_sc:].reshape(n_tc // blk, 1, blk)
    p_hi, q_lo, pt_hi, qt_lo = _trig_tables(pe)
    tc = _tc_trig_kernel(n_tc, d, blk, v // 64, 64)
    out_tc = tc(idx_tc, p_hi, q_lo, pt_hi, qt_lo)

    return jnp.concatenate(
        [out_sc.reshape(b_sc, s, d), out_tc.reshape(b - b_sc, s, d)], axis=0
    )


# speedup vs baseline: 1.9655x; 1.9655x over previous
import functools

import jax
import jax.numpy as jnp
from jax import lax
from jax.experimental import pallas as pl
from jax.experimental.pallas import tpu as pltpu
from jax.experimental.pallas import tpu_sc as plsc

_NBUF = 6


def _gather_kernel(n_total, d_model, b_per_w, chunk, n_chunks):
    mesh = plsc.VectorSubcoreMesh(core_axis_name="c", subcore_axis_name="s")

    @functools.partial(
        pl.kernel,
        mesh=mesh,
        out_type=jax.ShapeDtypeStruct((n_total, d_model), jnp.float32),
        scratch_types=[
            pltpu.VMEM((n_chunks, chunk), jnp.int32),
            pltpu.VMEM((_NBUF, chunk, d_model), jnp.float32),
            pltpu.SemaphoreType.DMA((_NBUF,)),
            pltpu.SemaphoreType.DMA((_NBUF,)),
        ],
    )
    def k(table_hbm, idx_hbm, out_hbm, idx_v, rows_v, gsem, ssem):
        nc = plsc.get_sparse_core_info().num_cores
        wid = lax.axis_index("s") * nc + lax.axis_index("c")
        base = wid * b_per_w
        pltpu.sync_copy(idx_hbm.at[wid], idx_v)

        def gather(c, b):
            cp = pltpu.make_async_copy(
                table_hbm.at[idx_v.at[c]], rows_v.at[b], gsem.at[b]
            )
            cp.start()
            return cp

        def scatter(c, b):
            pltpu.make_async_copy(
                table_hbm.at[idx_v.at[c]], rows_v.at[b], gsem.at[b]
            ).wait()
            cp = pltpu.make_async_copy(
                rows_v.at[b],
                out_hbm.at[pl.ds(base + c * chunk, chunk)],
                ssem.at[b],
            )
            cp.start()
            return cp

        def wait_scatter(c, b):
            pltpu.make_async_copy(
                rows_v.at[b],
                out_hbm.at[pl.ds(base + c * chunk, chunk)],
                ssem.at[b],
            ).wait()

        for c in range(_NBUF):
            gather(c, c)
        scatter(0, 0)

        n_groups = (n_chunks - _NBUF) // _NBUF
        c_end = n_groups * _NBUF

        def body(j):
            c0 = 1 + j * _NBUF
            for u in range(_NBUF):
                c = c0 + u
                scatter(c, (1 + u) % _NBUF)
                wait_scatter(c - 1, u % _NBUF)
                gather(c + _NBUF - 1, u % _NBUF)

        pl.loop(0, n_groups)(body)

        for c in range(c_end + 1, n_chunks):
            scatter(c, c % _NBUF)
            wait_scatter(c - 1, (c - 1) % _NBUF)
            if c + _NBUF - 1 < n_chunks:
                gather(c + _NBUF - 1, (c - 1) % _NBUF)
        wait_scatter(n_chunks - 1, (n_chunks - 1) % _NBUF)

    return k


def kernel(x, pe):
    b, s = x.shape
    v, d = pe.shape
    n = b * s
    info = plsc.get_sparse_core_info()
    nw = info.num_cores * info.num_subcores
    b_per_w = n // nw
    chunk = 16
    n_chunks = b_per_w // chunk
    idx3 = x.astype(jnp.int32).reshape(nw, n_chunks, chunk)
    k = _gather_kernel(n, d, b_per_w, chunk, n_chunks)
    out = k(pe, idx3)
    return out.reshape(b, s, d)
